# baseline (speedup 1.0000x reference)
import functools

import jax
import jax.numpy as jnp
from jax import lax
from jax.experimental import pallas as pl
from jax.experimental.pallas import tpu as pltpu
from jax.experimental.pallas import tpu_sc as plsc

_VOCAB = 1000000
_EMB = 32
_BATCH = 4096
_MAXLEN = 50

_NC = 2
_NS = 16
_NW = _NC * _NS

_CHUNK = 80
_ROWS_PER_PAIR = _BATCH * _MAXLEN
_CHUNKS_PER_PAIR = _ROWS_PER_PAIR // _CHUNK
_CPW = _CHUNKS_PER_PAIR // _NW
_NB = 10
_NGROUPS = _CPW // _NB


def _gather_body(table, idx, out0, out1, idx_v, bufs, gsem, wsem):
    wid = lax.axis_index("s") * _NC + lax.axis_index("c")
    pltpu.sync_copy(idx.at[pl.ds(wid * _CPW, _CPW)], idx_v.at[0])
    pltpu.sync_copy(
        idx.at[pl.ds(_CHUNKS_PER_PAIR + wid * _CPW, _CPW)], idx_v.at[1]
    )
    for p, out in ((0, out0), (1, out1)):
        def group(g, carry):
            @pl.when(g > 0)
            def _():
                for b in range(_NB):
                    pltpu.make_async_copy(
                        bufs.at[b], out.at[pl.ds(0, _CHUNK)], wsem
                    ).wait()

            for b in range(_NB):
                jj = g * _NB + b
                pltpu.async_copy(table.at[idx_v.at[p, jj]], bufs.at[b], gsem)
            for b in range(_NB):
                jj = g * _NB + b
                pltpu.make_async_copy(
                    table.at[idx_v.at[p, jj]], bufs.at[b], gsem
                ).wait()
            for b in range(_NB):
                jj = g * _NB + b
                row = (wid * _CPW + jj) * _CHUNK
                pltpu.async_copy(bufs.at[b], out.at[pl.ds(row, _CHUNK)], wsem)
            return carry

        lax.fori_loop(0, _NGROUPS, group, 0)
        for b in range(_NB):
            pltpu.make_async_copy(
                bufs.at[b], out.at[pl.ds(0, _CHUNK)], wsem
            ).wait()


_gather = pl.kernel(
    _gather_body,
    out_type=(
        jax.ShapeDtypeStruct((_ROWS_PER_PAIR, _EMB), jnp.float32),
        jax.ShapeDtypeStruct((_ROWS_PER_PAIR, _EMB), jnp.float32),
    ),
    mesh=plsc.VectorSubcoreMesh(core_axis_name="c", subcore_axis_name="s"),
    compiler_params=pltpu.CompilerParams(use_tc_tiling_on_sc=False),
    scratch_types=[
        pltpu.VMEM((2, _CPW, _CHUNK), jnp.int32),
        pltpu.VMEM((_NB, _CHUNK, _EMB), jnp.float32),
        pltpu.SemaphoreType.DMA,
        pltpu.SemaphoreType.DMA,
    ],
)


def _seqlen_body(x_ref, o_ref):
    o_ref[...] = jnp.sum(jnp.sign(x_ref[...]), axis=1, keepdims=True)


_seqlen = pl.pallas_call(
    _seqlen_body,
    out_shape=jax.ShapeDtypeStruct((_BATCH * 2, 1), jnp.int32),
)


def kernel(x, embedding_w):
    idx2 = jnp.transpose(x, (1, 0, 2)).reshape(2 * _CHUNKS_PER_PAIR, _CHUNK)
    out0, out1 = _gather(embedding_w, idx2)
    e_hypo = out0.reshape(_BATCH, _MAXLEN, _EMB)
    e_prem = out1.reshape(_BATCH, _MAXLEN, _EMB)
    seq = _seqlen(x.reshape(_BATCH * 2, _MAXLEN)).reshape(_BATCH, 2)
    return (e_hypo, e_prem, seq[:, 0], seq[:, 1])
